# baseline (speedup 1.0000x reference)
import functools

import jax
import jax.numpy as jnp
from jax.experimental import pallas as pl
from jax.experimental.pallas import tpu as pltpu

_EPS = 1e-5
_NEG_INF = -1e9


def _rmsnorm(x, w):
    return x * jax.lax.rsqrt(jnp.mean(x * x, axis=-1, keepdims=True) + _EPS) * w


def _qkv_body(x_ref, n1_ref, wq_ref, wk_ref, wv_ref, scale_ref,
              q_ref, k_ref, v_ref):
    xn = _rmsnorm(x_ref[...], n1_ref[...])
    q_ref[...] = jnp.dot(xn, wq_ref[...],
                         preferred_element_type=jnp.float32) * scale_ref[0, 0]
    k_ref[...] = jnp.dot(xn, wk_ref[...], preferred_element_type=jnp.float32)
    v_ref[...] = jnp.dot(xn, wv_ref[...], preferred_element_type=jnp.float32)


def _qkv(x2d, norm1_w, wq, wk, wv, scale2d):
    b, dim = x2d.shape
    vspec = pl.BlockSpec(memory_space=pltpu.VMEM)
    return pl.pallas_call(
        _qkv_body,
        in_specs=[vspec, vspec, vspec, vspec, vspec,
                  pl.BlockSpec(memory_space=pltpu.SMEM)],
        out_specs=[vspec, vspec, vspec],
        out_shape=[jax.ShapeDtypeStruct((b, dim), jnp.float32)] * 3,
    )(x2d, norm1_w, wq, wk, wv, scale2d)


def _attn_body(bt_ref, ctx_ref, slot_ref,
               q_ref, k_page_ref, v_page_ref, knew_ref, vnew_ref,
               o_ref, m_scr, l_scr, acc_scr, *, bs, maxb, nbatch):
    b = pl.program_id(0)
    p = pl.program_id(1)
    ctx = ctx_ref[b]
    npages = (ctx + bs - 1) // bs

    @pl.when(p == 0)
    def _init():
        m_scr[...] = jnp.full_like(m_scr, -1e30)
        l_scr[...] = jnp.zeros_like(l_scr)
        acc_scr[...] = jnp.zeros_like(acc_scr)

    @pl.when(p < npages)
    def _compute():
        page_id = bt_ref[b, p]
        for b2 in range(nbatch):
            blk2 = slot_ref[b2] // bs

            @pl.when(blk2 == page_id)
            def _patch(b2=b2):
                off2 = slot_ref[b2] % bs
                row = jax.lax.broadcasted_iota(jnp.int32, (1, bs, 1), 1) == off2
                k_page_ref[0] = jnp.where(
                    row, knew_ref[b2][:, None, :], k_page_ref[0])
                v_page_ref[0] = jnp.where(
                    row, vnew_ref[b2][:, None, :], v_page_ref[0])

        q = q_ref[0]
        k_page = k_page_ref[0]
        v_page = v_page_ref[0]
        s = jnp.sum(q[:, None, :] * k_page, axis=-1)
        pos = p * bs + jax.lax.broadcasted_iota(jnp.int32, s.shape, 1)
        s = jnp.where(pos < ctx, s, _NEG_INF)

        m_old = m_scr[...]
        m_new = jnp.maximum(m_old, jnp.max(s, axis=-1, keepdims=True))
        alpha = jnp.exp(m_old - m_new)
        pexp = jnp.exp(s - m_new[:, 0:1])
        m_scr[...] = m_new
        l_scr[...] = l_scr[...] * alpha + jnp.sum(pexp, axis=-1, keepdims=True)
        acc_scr[...] = (acc_scr[...] * alpha[:, 0:1]
                        + jnp.sum(pexp[:, :, None] * v_page, axis=1))

    @pl.when(p == maxb - 1)
    def _finish():
        o_ref[0] = acc_scr[...] / l_scr[:, 0:1]


def _attn(q3d, key_heap, val_heap, knew3d, vnew3d,
          block_table, context_lens, slot_mapping):
    b, h, hd = q3d.shape
    nblk, _, bs, _ = key_heap.shape
    maxb = block_table.shape[1]

    def kv_index_map(bi, pi, bt, ctx, slot):
        npages = (ctx[bi] + bs - 1) // bs
        p_eff = jnp.minimum(pi, npages - 1)
        return (bt[bi, p_eff], 0, 0, 0)

    grid_spec = pltpu.PrefetchScalarGridSpec(
        num_scalar_prefetch=3,
        grid=(b, maxb),
        in_specs=[
            pl.BlockSpec((1, h, hd), lambda bi, pi, bt, ctx, slot: (bi, 0, 0)),
            pl.BlockSpec((1, h, bs, hd), kv_index_map),
            pl.BlockSpec((1, h, bs, hd), kv_index_map),
            pl.BlockSpec((b, h, hd), lambda bi, pi, bt, ctx, slot: (0, 0, 0)),
            pl.BlockSpec((b, h, hd), lambda bi, pi, bt, ctx, slot: (0, 0, 0)),
        ],
        out_specs=pl.BlockSpec((1, h, hd),
                               lambda bi, pi, bt, ctx, slot: (bi, 0, 0)),
        scratch_shapes=[
            pltpu.VMEM((h, 128), jnp.float32),
            pltpu.VMEM((h, 128), jnp.float32),
            pltpu.VMEM((h, hd), jnp.float32),
        ],
    )
    body = functools.partial(_attn_body, bs=bs, maxb=maxb, nbatch=b)
    return pl.pallas_call(
        body,
        grid_spec=grid_spec,
        out_shape=jax.ShapeDtypeStruct((b, h, hd), jnp.float32),
        compiler_params=pltpu.CompilerParams(
            dimension_semantics=("arbitrary", "arbitrary")),
    )(block_table, context_lens, slot_mapping,
      q3d, key_heap, val_heap, knew3d, vnew3d)


def _mlp_body(attn_ref, res_ref, wo_ref, n2_ref, w1_ref, w3_ref, w2_ref,
              o_ref, x2_scr, xn2_scr, acc_scr, *, nsteps):
    j = pl.program_id(0)

    @pl.when(j == 0)
    def _first():
        x2 = jnp.dot(attn_ref[...], wo_ref[...],
                     preferred_element_type=jnp.float32) + res_ref[...]
        x2_scr[...] = x2
        xn2_scr[...] = _rmsnorm(x2, n2_ref[...])
        acc_scr[...] = jnp.zeros_like(acc_scr)

    xn2 = xn2_scr[...]
    g = jnp.dot(xn2, w1_ref[...], preferred_element_type=jnp.float32)
    u = jnp.dot(xn2, w3_ref[...], preferred_element_type=jnp.float32)
    hc = g * (1.0 / (1.0 + jnp.exp(-g))) * u
    acc_scr[...] += jnp.dot(hc, w2_ref[...], preferred_element_type=jnp.float32)

    @pl.when(j == nsteps - 1)
    def _last():
        o_ref[...] = acc_scr[...] + x2_scr[...]


def _mlp(attn2d, res2d, wo, norm2_w, w1, w2, w3):
    b, dim = attn2d.shape
    hidden = w1.shape[1]
    nsteps = 4
    ch = hidden // nsteps
    full = pl.BlockSpec((b, dim), lambda j: (0, 0))
    wfull = pl.BlockSpec((dim, dim), lambda j: (0, 0))
    body = functools.partial(_mlp_body, nsteps=nsteps)
    return pl.pallas_call(
        body,
        grid=(nsteps,),
        in_specs=[
            full, full, wfull,
            pl.BlockSpec((1, dim), lambda j: (0, 0)),
            pl.BlockSpec((dim, ch), lambda j: (0, j)),
            pl.BlockSpec((dim, ch), lambda j: (0, j)),
            pl.BlockSpec((ch, dim), lambda j: (j, 0)),
        ],
        out_specs=full,
        out_shape=jax.ShapeDtypeStruct((b, dim), jnp.float32),
        scratch_shapes=[
            pltpu.VMEM((b, dim), jnp.float32),
            pltpu.VMEM((b, dim), jnp.float32),
            pltpu.VMEM((b, dim), jnp.float32),
        ],
        compiler_params=pltpu.CompilerParams(
            dimension_semantics=("arbitrary",)),
    )(attn2d, res2d, wo, norm2_w, w1, w3, w2)


def kernel(x, key_heap, val_heap, block_table, slot_mapping, context_lens,
           exp_sums, max_logits, tmp_output, scale, k_scale, v_scale,
           max_seq_len, wq, wk, wv, wo, norm1_w, norm2_w, w1, w2, w3):
    b, _, dim = x.shape
    _, h, bs, hd = key_heap.shape

    x2d = x.reshape(b, dim)
    scale2d = jnp.asarray(scale, jnp.float32).reshape(1, 1)
    qs, kn, vn = _qkv(x2d, norm1_w.reshape(1, dim), wq, wk, wv, scale2d)

    attn = _attn(qs.reshape(b, h, hd), key_heap, val_heap,
                 kn.reshape(b, h, hd), vn.reshape(b, h, hd),
                 block_table, context_lens, slot_mapping)

    out = _mlp(attn.reshape(b, dim), x2d, wo, norm2_w.reshape(1, dim),
               w1, w2, w3)
    return out.reshape(b, 1, dim)
